# baseline (speedup 1.0000x reference)
import functools

import jax
import jax.numpy as jnp
from jax import lax
from jax.experimental import pallas as pl
from jax.experimental.pallas import tpu as pltpu
from jax.experimental.pallas import tpu_sc as plsc

B = 4096
DAY = 7
AF = 26
UF = 26
EMB = 16
AP = DAY * AF
DP = 8
LP = 128
GW = 32
NW = 32
RPW = B // NW
CH = 8
NCHUNK = RPW // CH
GPC = CH * (DAY + 1)


_sc_mesh = plsc.VectorSubcoreMesh(core_axis_name="c", subcore_axis_name="s")


@functools.partial(
    pl.kernel,
    out_type=jax.ShapeDtypeStruct((B, EMB), jnp.float32),
    mesh=_sc_mesh,
    compiler_params=pltpu.CompilerParams(use_tc_tiling_on_sc=False),
    scratch_types=[
        pltpu.VMEM((CH * DP * LP,), jnp.int32),
        pltpu.VMEM((CH * LP,), jnp.int32),
        pltpu.VMEM((CH * DP * LP,), jnp.float32),
        pltpu.VMEM((CH * LP,), jnp.float32),
        pltpu.VMEM((CH * DAY * GW, EMB), jnp.float32),
        pltpu.VMEM((CH * GW, EMB), jnp.float32),
        pltpu.VMEM((AP * EMB,), jnp.float32),
        pltpu.VMEM((UF * EMB,), jnp.float32),
        pltpu.VMEM((RPW, EMB), jnp.float32),
        pltpu.SemaphoreType.DMA,
    ],
)
def _sc_gather_dot(a_table, u_table, ai, av, ui, uv, wa, wu, zp_hbm,
                   aidx_v, uidx_v, av_v, uv_v, arow_v, urow_v,
                   wa_v, wu_v, zp_v, sem):
    wid = lax.axis_index("s") * 2 + lax.axis_index("c")
    base = wid * RPW

    pltpu.sync_copy(wa, wa_v)
    pltpu.sync_copy(wu, wu_v)

    def chunk_body(c, carry):
        row0 = base + c * CH
        pltpu.sync_copy(ai.at[pl.ds(row0 * DP * LP, CH * DP * LP)], aidx_v)
        pltpu.sync_copy(ui.at[pl.ds(row0 * LP, CH * LP)], uidx_v)
        pltpu.sync_copy(av.at[pl.ds(row0 * DP * LP, CH * DP * LP)], av_v)
        pltpu.sync_copy(uv.at[pl.ds(row0 * LP, CH * LP)], uv_v)

        def fire_body(r, carry2):
            for d in range(DAY):
                pltpu.async_copy(
                    a_table.at[aidx_v.at[pl.ds((r * DP + d) * LP, GW)]],
                    arow_v.at[pl.ds((r * DAY + d) * GW, GW)], sem)
            pltpu.async_copy(
                u_table.at[uidx_v.at[pl.ds(r * LP, GW)]],
                urow_v.at[pl.ds(r * GW, GW)], sem)
            return carry2

        lax.fori_loop(0, CH, fire_body, 0)

        drain = pltpu.make_async_copy(
            u_table.at[uidx_v.at[pl.ds(0, GW)]],
            urow_v.at[pl.ds(0, GW)], sem)

        def drain_body(i, carry2):
            drain.wait()
            return carry2

        lax.fori_loop(0, GPC, drain_body, 0)

        def row_body(r, carry2):
            def day_group(d, acc):
                rbase = (r * DAY + d) * GW
                vbase = (r * DP + d) * LP
                vals = av_v[pl.ds(vbase, 16)]
                for j in range(16):
                    row = arow_v[rbase + j, :]
                    wv = wa_v[pl.ds((d * AF + j) * 16, 16)]
                    acc = acc + row * (wv * vals[j])
                tvals = av_v[pl.ds(vbase + AF - 16, 16)]
                for j in range(AF - 16):
                    f = 16 + j
                    row = arow_v[rbase + f, :]
                    wv = wa_v[pl.ds((d * AF + f) * 16, 16)]
                    acc = acc + row * (wv * tvals[16 - (AF - 16) + j])
                return acc

            acc = lax.fori_loop(0, DAY, day_group,
                                jnp.zeros((16,), jnp.float32))

            ubase = r * GW
            uvals = uv_v[pl.ds(r * LP, 16)]
            for j in range(16):
                row = urow_v[ubase + j, :]
                wv = wu_v[pl.ds(j * 16, 16)]
                acc = acc + row * (wv * uvals[j])
            utvals = uv_v[pl.ds(r * LP + UF - 16, 16)]
            for j in range(UF - 16):
                f = 16 + j
                row = urow_v[ubase + f, :]
                wv = wu_v[pl.ds(f * 16, 16)]
                acc = acc + row * (wv * utvals[16 - (UF - 16) + j])

            zp_v[c * CH + r, :] = acc
            return carry2

        lax.fori_loop(0, CH, row_body, 0)
        return carry

    lax.fori_loop(0, NCHUNK, chunk_body, 0)
    pltpu.sync_copy(zp_v, zp_hbm.at[pl.ds(base, RPW)])


def _epi_body(zp_ref, y_ref, b_ref, out_ref, loss_ref):
    z = jnp.sum(zp_ref[...], axis=1, keepdims=True) + b_ref[0, 0]
    out = 1.0 / (1.0 + jnp.exp(-z))
    out_ref[...] = out
    yb = (y_ref[...] >= 1e-5).astype(jnp.float32)
    p = jnp.clip(out, 1e-7, 1.0 - 1e-7)
    loss_ref[0, 0] = jnp.mean(-(yb * jnp.log(p) + (1.0 - yb) * jnp.log(1.0 - p)))


_epilogue = pl.pallas_call(
    _epi_body,
    out_shape=(jax.ShapeDtypeStruct((B, 1), jnp.float32),
               jax.ShapeDtypeStruct((1, 1), jnp.float32)),
    in_specs=[pl.BlockSpec(memory_space=pltpu.VMEM),
              pl.BlockSpec(memory_space=pltpu.VMEM),
              pl.BlockSpec(memory_space=pltpu.SMEM)],
    out_specs=(pl.BlockSpec(memory_space=pltpu.VMEM),
               pl.BlockSpec(memory_space=pltpu.SMEM)),
)


def kernel(ui, uv, ai, av, y, a_table, u_table, W, b):
    ai_p = jnp.pad(ai.astype(jnp.int32),
                   ((0, 0), (0, DP - DAY), (0, LP - AF))).reshape(-1)
    av_p = jnp.pad(av, ((0, 0), (0, DP - DAY), (0, LP - AF))).reshape(-1)
    ui_p = jnp.pad(ui.astype(jnp.int32), ((0, 0), (0, LP - UF))).reshape(-1)
    uv_p = jnp.pad(uv, ((0, 0), (0, LP - UF))).reshape(-1)
    wa = W[:AP * EMB, 0]
    wu = W[AP * EMB:, 0]

    zp = _sc_gather_dot(a_table, u_table, ai_p, av_p, ui_p, uv_p, wa, wu)
    out, loss = _epilogue(zp, y, b.reshape(1, 1))
    return (loss.reshape(()), out)


# speedup vs baseline: 1.8355x; 1.8355x over previous
import functools

import jax
import jax.numpy as jnp
from jax import lax
from jax.experimental import pallas as pl
from jax.experimental.pallas import tpu as pltpu
from jax.experimental.pallas import tpu_sc as plsc

B = 4096
DAY = 7
AF = 26
UF = 26
EMB = 16
AP = DAY * AF
NW = 32
RPW = B // NW
CH = 8
NCHUNK = RPW // CH
AIC = CH * AP
UIC = CH * UF
GD = 128


_sc_mesh = plsc.VectorSubcoreMesh(core_axis_name="c", subcore_axis_name="s")


@functools.partial(
    pl.kernel,
    out_type=jax.ShapeDtypeStruct((B, EMB), jnp.float32),
    mesh=_sc_mesh,
    compiler_params=pltpu.CompilerParams(use_tc_tiling_on_sc=False),
    scratch_types=[
        pltpu.VMEM((AIC,), jnp.int32),
        pltpu.VMEM((UIC,), jnp.int32),
        pltpu.VMEM((AIC,), jnp.float32),
        pltpu.VMEM((UIC,), jnp.float32),
        pltpu.VMEM((AIC, EMB), jnp.float32),
        pltpu.VMEM((UIC, EMB), jnp.float32),
        pltpu.VMEM((AP * EMB,), jnp.float32),
        pltpu.VMEM((UF * EMB,), jnp.float32),
        pltpu.VMEM((RPW, EMB), jnp.float32),
        pltpu.SemaphoreType.DMA,
    ],
)
def _sc_gather_dot(a_table, u_table, ai, av, ui, uv, wa, wu, zp_hbm,
                   aidx_v, uidx_v, av_v, uv_v, arow_v, urow_v,
                   wa_v, wu_v, zp_v, sem):
    wid = lax.axis_index("s") * 2 + lax.axis_index("c")
    base = wid * RPW

    pltpu.sync_copy(wa, wa_v)
    pltpu.sync_copy(wu, wu_v)

    def chunk_body(c, carry):
        row0 = base + c * CH
        pltpu.sync_copy(ai.at[pl.ds(row0 * AP, AIC)], aidx_v)
        pltpu.sync_copy(ui.at[pl.ds(row0 * UF, UIC)], uidx_v)
        pltpu.sync_copy(av.at[pl.ds(row0 * AP, AIC)], av_v)
        pltpu.sync_copy(uv.at[pl.ds(row0 * UF, UIC)], uv_v)

        copies = []
        for j in range(AIC // GD):
            copies.append(pltpu.async_copy(
                a_table.at[aidx_v.at[pl.ds(j * GD, GD)]],
                arow_v.at[pl.ds(j * GD, GD)], sem))
        a_rem = AIC % GD
        copies.append(pltpu.async_copy(
            a_table.at[aidx_v.at[pl.ds(AIC - a_rem, a_rem)]],
            arow_v.at[pl.ds(AIC - a_rem, a_rem)], sem))
        copies.append(pltpu.async_copy(
            u_table.at[uidx_v.at[pl.ds(0, GD)]],
            urow_v.at[pl.ds(0, GD)], sem))
        u_rem = UIC - GD
        copies.append(pltpu.async_copy(
            u_table.at[uidx_v.at[pl.ds(GD, u_rem)]],
            urow_v.at[pl.ds(GD, u_rem)], sem))
        for cp in copies:
            cp.wait()

        def row_body(r, carry2):
            def a_pg(pg, acc):
                vbase = r * AP + pg * 16
                vals = av_v[pl.ds(vbase, 16)]
                for j in range(16):
                    row = arow_v[vbase + j, :]
                    wv = wa_v[pl.ds((pg * 16 + j) * 16, 16)]
                    acc = acc + row * (wv * vals[j])
                return acc

            acc = lax.fori_loop(0, AP // 16, a_pg,
                                jnp.zeros((16,), jnp.float32))

            tbase = r * AP + AP - 16
            tvals = av_v[pl.ds(tbase, 16)]
            for j in range(AP % 16):
                p = (AP // 16) * 16 + j
                row = arow_v[r * AP + p, :]
                wv = wa_v[pl.ds(p * 16, 16)]
                acc = acc + row * (wv * tvals[16 - (AP % 16) + j])

            vbase = r * UF
            uvals = uv_v[pl.ds(vbase, 16)]
            for j in range(16):
                row = urow_v[vbase + j, :]
                wv = wu_v[pl.ds(j * 16, 16)]
                acc = acc + row * (wv * uvals[j])

            utbase = r * UF + UF - 16
            utvals = uv_v[pl.ds(utbase, 16)]
            for j in range(UF - 16):
                p = 16 + j
                row = urow_v[r * UF + p, :]
                wv = wu_v[pl.ds(p * 16, 16)]
                acc = acc + row * (wv * utvals[16 - (UF - 16) + j])

            zp_v[c * CH + r, :] = acc
            return carry2

        lax.fori_loop(0, CH, row_body, 0)
        return carry

    lax.fori_loop(0, NCHUNK, chunk_body, 0)
    pltpu.sync_copy(zp_v, zp_hbm.at[pl.ds(base, RPW)])


def _epi_body(zp_ref, y_ref, b_ref, out_ref, loss_ref):
    z = jnp.sum(zp_ref[...], axis=1, keepdims=True) + b_ref[0, 0]
    out = 1.0 / (1.0 + jnp.exp(-z))
    out_ref[...] = out
    yb = (y_ref[...] >= 1e-5).astype(jnp.float32)
    p = jnp.clip(out, 1e-7, 1.0 - 1e-7)
    loss_ref[0, 0] = jnp.mean(-(yb * jnp.log(p) + (1.0 - yb) * jnp.log(1.0 - p)))


_epilogue = pl.pallas_call(
    _epi_body,
    out_shape=(jax.ShapeDtypeStruct((B, 1), jnp.float32),
               jax.ShapeDtypeStruct((1, 1), jnp.float32)),
    in_specs=[pl.BlockSpec(memory_space=pltpu.VMEM),
              pl.BlockSpec(memory_space=pltpu.VMEM),
              pl.BlockSpec(memory_space=pltpu.SMEM)],
    out_specs=(pl.BlockSpec(memory_space=pltpu.VMEM),
               pl.BlockSpec(memory_space=pltpu.SMEM)),
)


def kernel(ui, uv, ai, av, y, a_table, u_table, W, b):
    ai_f = ai.astype(jnp.int32).reshape(-1)
    av_f = av.reshape(-1)
    ui_f = ui.astype(jnp.int32).reshape(-1)
    uv_f = uv.reshape(-1)
    wa = W[:AP * EMB, 0]
    wu = W[AP * EMB:, 0]

    zp = _sc_gather_dot(a_table, u_table,
                        ai_f, av_f, ui_f, uv_f, wa, wu)
    out, loss = _epilogue(zp, y, b.reshape(1, 1))
    return (loss.reshape(()), out)
